# baseline (device time: 167740 ns/iter reference)
import jax
import jax.numpy as jnp
from jax import lax
from jax.experimental import pallas as pl
from jax.experimental.pallas import tpu as pltpu

N_DEV = 4
SQ = 1024
SKV = 1024
HQ = 8
DH = 128
SCALE = 0.08838834764831843


def _attn_body(q_ref, k_ref, v_ref, out_ref,
               kcomm, vcomm, acc_v, acc_s,
               ksend, krecv, vsend, vrecv):
    my = lax.axis_index("i")
    left = (my - 1) % N_DEV
    right = (my + 1) % N_DEV

    barrier_sem = pltpu.get_barrier_semaphore()
    for nbr in (left, right):
        pl.semaphore_signal(barrier_sem, inc=1, device_id=(nbr,),
                            device_id_type=pl.DeviceIdType.MESH)
    pl.semaphore_wait(barrier_sem, 2)

    bi = lax.broadcasted_iota(jnp.int32, (SQ, SKV), 0)
    bj = lax.broadcasted_iota(jnp.int32, (SQ, SKV), 1)
    mask = ((bi // 64) % 4) == ((bj // 64) % 4)

    acc_v[...] = jnp.zeros((HQ, SQ, DH), jnp.float32)
    acc_s[...] = jnp.zeros((HQ, SQ, 1), jnp.float32)

    def chunk_compute(kc_ref, vc_ref):
        def head_body(h, carry):
            qh = q_ref[h]
            kh = kc_ref[h]
            vh = vc_ref[h]
            s = lax.dot_general(qh, kh, (((1,), (1,)), ((), ())),
                                preferred_element_type=jnp.float32) * SCALE
            w = jnp.where(mask, jnp.exp(s), 0.0)
            pv = lax.dot_general(w.astype(jnp.bfloat16), vh,
                                 (((1,), (0,)), ((), ())),
                                 preferred_element_type=jnp.float32)
            acc_v[h] = acc_v[h] + pv
            acc_s[h] = acc_s[h] + jnp.sum(w, axis=1, keepdims=True)
            return carry
        lax.fori_loop(0, HQ, head_body, 0)

    for c in range(N_DEV):
        if c == 0:
            kc, vc = k_ref, v_ref
        else:
            kc, vc = kcomm.at[(c - 1) % 2], vcomm.at[(c - 1) % 2]
        if c < N_DEV - 1:
            slot = c % 2
            rk = pltpu.make_async_remote_copy(
                src_ref=kc, dst_ref=kcomm.at[slot],
                send_sem=ksend.at[slot], recv_sem=krecv.at[slot],
                device_id=(right,), device_id_type=pl.DeviceIdType.MESH)
            rv = pltpu.make_async_remote_copy(
                src_ref=vc, dst_ref=vcomm.at[slot],
                send_sem=vsend.at[slot], recv_sem=vrecv.at[slot],
                device_id=(right,), device_id_type=pl.DeviceIdType.MESH)
            rk.start()
            rv.start()
        chunk_compute(kc, vc)
        if c < N_DEV - 1:
            rk.wait()
            rv.wait()

    def store_body(h, carry):
        out_ref[:, pl.ds(h * DH, DH)] = acc_v[h] / acc_s[h]
        return carry
    lax.fori_loop(0, HQ, store_body, 0)


def kernel(x, Wq, K_ext, V_ext, Wo):
    Q = x[0] @ Wq
    Qt = jnp.transpose(Q.reshape(SQ, HQ, DH), (1, 0, 2)).astype(jnp.bfloat16)
    Kt = jnp.transpose(K_ext[0], (1, 0, 2)).astype(jnp.bfloat16)
    Vt = jnp.transpose(V_ext[0], (1, 0, 2)).astype(jnp.bfloat16)

    ctx = pl.pallas_call(
        _attn_body,
        out_shape=jax.ShapeDtypeStruct((SQ, HQ * DH), jnp.float32),
        in_specs=[pl.BlockSpec(memory_space=pltpu.VMEM)] * 3,
        out_specs=pl.BlockSpec(memory_space=pltpu.VMEM),
        scratch_shapes=[
            pltpu.VMEM((2, HQ, SKV, DH), jnp.bfloat16),
            pltpu.VMEM((2, HQ, SKV, DH), jnp.bfloat16),
            pltpu.VMEM((HQ, SQ, DH), jnp.float32),
            pltpu.VMEM((HQ, SQ, 1), jnp.float32),
            pltpu.SemaphoreType.DMA((2,)),
            pltpu.SemaphoreType.DMA((2,)),
            pltpu.SemaphoreType.DMA((2,)),
            pltpu.SemaphoreType.DMA((2,)),
        ],
        compiler_params=pltpu.CompilerParams(collective_id=0),
    )(Qt, Kt, Vt)

    return (ctx @ Wo)[None]


# device time: 103155 ns/iter; 1.6261x vs baseline; 1.6261x over previous
import jax
import jax.numpy as jnp
from jax import lax
from jax.experimental import pallas as pl
from jax.experimental.pallas import tpu as pltpu

N_DEV = 4
SQ = 1024
SKV = 1024
HKV = SKV // 2
HQ = 8
DH = 128
SCALE = 0.08838834764831843


def _attn_body(q_ref, k_ref, v_ref, out_ref,
               kL, vL, kR, vR, kO, vO, acc_v, acc_s, S, R):
    my = lax.axis_index("i")
    left = (my - 1) % N_DEV
    right = (my + 1) % N_DEV

    barrier_sem = pltpu.get_barrier_semaphore()
    for nbr in (left, right):
        pl.semaphore_signal(barrier_sem, inc=1, device_id=(nbr,),
                            device_id_type=pl.DeviceIdType.MESH)
    pl.semaphore_wait(barrier_sem, 2)

    bi = lax.broadcasted_iota(jnp.int32, (SQ, HKV), 0)
    bj = lax.broadcasted_iota(jnp.int32, (SQ, HKV), 1)
    mask = ((bi // 64) % 4) == ((bj // 64) % 4)

    a = [
        pltpu.make_async_remote_copy(k_ref, kL, S.at[0], R.at[0],
                                     device_id=(right,),
                                     device_id_type=pl.DeviceIdType.MESH),
        pltpu.make_async_remote_copy(v_ref, vL, S.at[1], R.at[1],
                                     device_id=(right,),
                                     device_id_type=pl.DeviceIdType.MESH),
        pltpu.make_async_remote_copy(k_ref, kR, S.at[2], R.at[2],
                                     device_id=(left,),
                                     device_id_type=pl.DeviceIdType.MESH),
        pltpu.make_async_remote_copy(v_ref, vR, S.at[3], R.at[3],
                                     device_id=(left,),
                                     device_id_type=pl.DeviceIdType.MESH),
    ]
    for r in a:
        r.start()

    acc_v[...] = jnp.zeros((HQ, SQ, DH), jnp.float32)
    acc_s[...] = jnp.zeros((HQ, SQ, 1), jnp.float32)

    def half_compute(kc, vc):
        def head_body(h, carry):
            s = lax.dot_general(q_ref[h], kc[h], (((1,), (1,)), ((), ())),
                                preferred_element_type=jnp.float32) * SCALE
            w = jnp.where(mask, jnp.exp(s), 0.0)
            pv = lax.dot_general(w.astype(jnp.bfloat16), vc[h],
                                 (((1,), (0,)), ((), ())),
                                 preferred_element_type=jnp.float32)
            acc_v[h] = acc_v[h] + pv
            acc_s[h] = acc_s[h] + jnp.sum(w, axis=1, keepdims=True)
            return carry
        lax.fori_loop(0, HQ, head_body, 0)

    half_compute(k_ref.at[0], v_ref.at[0])
    half_compute(k_ref.at[1], v_ref.at[1])

    a[0].wait_recv()
    a[1].wait_recv()
    b_right = [
        pltpu.make_async_remote_copy(kL.at[1], kO.at[1], S.at[4], R.at[6],
                                     device_id=(right,),
                                     device_id_type=pl.DeviceIdType.MESH),
        pltpu.make_async_remote_copy(vL.at[1], vO.at[1], S.at[5], R.at[7],
                                     device_id=(right,),
                                     device_id_type=pl.DeviceIdType.MESH),
    ]
    for r in b_right:
        r.start()

    a[2].wait_recv()
    a[3].wait_recv()
    b_left = [
        pltpu.make_async_remote_copy(kR.at[0], kO.at[0], S.at[6], R.at[4],
                                     device_id=(left,),
                                     device_id_type=pl.DeviceIdType.MESH),
        pltpu.make_async_remote_copy(vR.at[0], vO.at[0], S.at[7], R.at[5],
                                     device_id=(left,),
                                     device_id_type=pl.DeviceIdType.MESH),
    ]
    for r in b_left:
        r.start()

    half_compute(kL.at[0], vL.at[0])
    half_compute(kL.at[1], vL.at[1])
    half_compute(kR.at[0], vR.at[0])
    half_compute(kR.at[1], vR.at[1])

    b_left[0].wait_recv()
    b_left[1].wait_recv()
    half_compute(kO.at[0], vO.at[0])
    b_right[0].wait_recv()
    b_right[1].wait_recv()
    half_compute(kO.at[1], vO.at[1])

    for r in a + b_right + b_left:
        r.wait_send()

    def store_body(h, carry):
        out_ref[:, pl.ds(h * DH, DH)] = acc_v[h] / acc_s[h]
        return carry
    lax.fori_loop(0, HQ, store_body, 0)


def kernel(x, Wq, K_ext, V_ext, Wo):
    Q = x[0] @ Wq
    Qt = jnp.transpose(Q.reshape(SQ, HQ, DH), (1, 0, 2)).astype(jnp.bfloat16)
    Kt = jnp.transpose(
        K_ext[0].reshape(2, HKV, HQ, DH), (0, 2, 1, 3)).astype(jnp.bfloat16)
    Vt = jnp.transpose(
        V_ext[0].reshape(2, HKV, HQ, DH), (0, 2, 1, 3)).astype(jnp.bfloat16)

    chunk = pltpu.VMEM((2, HQ, HKV, DH), jnp.bfloat16)
    ctx = pl.pallas_call(
        _attn_body,
        out_shape=jax.ShapeDtypeStruct((SQ, HQ * DH), jnp.float32),
        in_specs=[pl.BlockSpec(memory_space=pltpu.VMEM)] * 3,
        out_specs=pl.BlockSpec(memory_space=pltpu.VMEM),
        scratch_shapes=[
            chunk, chunk,
            chunk, chunk,
            chunk, chunk,
            pltpu.VMEM((HQ, SQ, DH), jnp.float32),
            pltpu.VMEM((HQ, SQ, 1), jnp.float32),
            pltpu.SemaphoreType.DMA((8,)),
            pltpu.SemaphoreType.DMA((8,)),
        ],
        compiler_params=pltpu.CompilerParams(collective_id=0),
    )(Qt, Kt, Vt)

    return (ctx @ Wo)[None]


# device time: 96942 ns/iter; 1.7303x vs baseline; 1.0641x over previous
import jax
import jax.numpy as jnp
from jax import lax
from jax.experimental import pallas as pl
from jax.experimental.pallas import tpu as pltpu

N_DEV = 4
SQ = 1024
SKV = 1024
QKV = SKV // 4
QQ = SQ // 4
HQ = 8
DH = 128
SCALE = 0.08838834764831843

_MESH = dict(device_id_type=pl.DeviceIdType.MESH)


def _attn_body(x_ref, wq_ref, k_ref, v_ref, out_ref,
               kL, vL, kR, vR, kO, vO, q_scr, acc_v, acc_s, S, R):
    my = lax.axis_index("i")
    left = (my - 1) % N_DEV
    right = (my + 1) % N_DEV

    barrier_sem = pltpu.get_barrier_semaphore()
    for nbr in (left, right):
        pl.semaphore_signal(barrier_sem, inc=1, device_id=(nbr,), **_MESH)
    pl.semaphore_wait(barrier_sem, 2)

    a_r = [
        pltpu.make_async_remote_copy(k_ref.at[pl.ds(2, 2)], kL.at[pl.ds(2, 2)],
                                     S.at[0], R.at[0], device_id=(right,), **_MESH),
        pltpu.make_async_remote_copy(v_ref.at[pl.ds(2, 2)], vL.at[pl.ds(2, 2)],
                                     S.at[1], R.at[1], device_id=(right,), **_MESH),
        pltpu.make_async_remote_copy(k_ref.at[pl.ds(0, 2)], kL.at[pl.ds(0, 2)],
                                     S.at[2], R.at[2], device_id=(right,), **_MESH),
        pltpu.make_async_remote_copy(v_ref.at[pl.ds(0, 2)], vL.at[pl.ds(0, 2)],
                                     S.at[3], R.at[3], device_id=(right,), **_MESH),
    ]
    a_l = [
        pltpu.make_async_remote_copy(k_ref.at[pl.ds(0, 2)], kR.at[pl.ds(0, 2)],
                                     S.at[4], R.at[4], device_id=(left,), **_MESH),
        pltpu.make_async_remote_copy(v_ref.at[pl.ds(0, 2)], vR.at[pl.ds(0, 2)],
                                     S.at[5], R.at[5], device_id=(left,), **_MESH),
        pltpu.make_async_remote_copy(k_ref.at[pl.ds(2, 2)], kR.at[pl.ds(2, 2)],
                                     S.at[6], R.at[6], device_id=(left,), **_MESH),
        pltpu.make_async_remote_copy(v_ref.at[pl.ds(2, 2)], vR.at[pl.ds(2, 2)],
                                     S.at[7], R.at[7], device_id=(left,), **_MESH),
    ]
    for r in a_r + a_l:
        r.start()

    def qproj_body(h, carry):
        q_scr[h] = lax.dot_general(
            x_ref[...], wq_ref[h], (((1,), (0,)), ((), ())),
            preferred_element_type=jnp.float32).astype(jnp.bfloat16)
        return carry
    lax.fori_loop(0, HQ, qproj_body, 0)

    acc_v[...] = jnp.zeros((HQ, SQ, DH), jnp.float32)
    acc_s[...] = jnp.zeros((HQ, SQ, 1), jnp.float32)

    def quarter(kc, vc, r):
        def head_body(h, carry):
            qh = q_scr[h, pl.ds(r * QQ, QQ)]
            s = lax.dot_general(qh, kc[h], (((1,), (1,)), ((), ())),
                                preferred_element_type=jnp.float32) * SCALE
            w = jnp.exp(s)
            pv = lax.dot_general(w.astype(jnp.bfloat16), vc[h],
                                 (((1,), (0,)), ((), ())),
                                 preferred_element_type=jnp.float32)
            row = pl.ds(r * QQ, QQ)
            acc_v[h, row] = acc_v[h, row] + pv
            acc_s[h, row] = acc_s[h, row] + jnp.sum(w, axis=1, keepdims=True)
            return carry
        lax.fori_loop(0, HQ, head_body, 0)

    for r in range(4):
        quarter(k_ref.at[r], v_ref.at[r], r)

    a_r[0].wait_recv()
    a_r[1].wait_recv()
    b_r = [
        pltpu.make_async_remote_copy(kL.at[2], kO.at[2], S.at[8], R.at[8],
                                     device_id=(right,), **_MESH),
        pltpu.make_async_remote_copy(vL.at[2], vO.at[2], S.at[9], R.at[9],
                                     device_id=(right,), **_MESH),
        pltpu.make_async_remote_copy(kL.at[3], kO.at[3], S.at[10], R.at[10],
                                     device_id=(right,), **_MESH),
        pltpu.make_async_remote_copy(vL.at[3], vO.at[3], S.at[11], R.at[11],
                                     device_id=(right,), **_MESH),
    ]
    for r in b_r:
        r.start()
    a_l[0].wait_recv()
    a_l[1].wait_recv()
    b_l = [
        pltpu.make_async_remote_copy(kR.at[0], kO.at[0], S.at[12], R.at[12],
                                     device_id=(left,), **_MESH),
        pltpu.make_async_remote_copy(vR.at[0], vO.at[0], S.at[13], R.at[13],
                                     device_id=(left,), **_MESH),
        pltpu.make_async_remote_copy(kR.at[1], kO.at[1], S.at[14], R.at[14],
                                     device_id=(left,), **_MESH),
        pltpu.make_async_remote_copy(vR.at[1], vO.at[1], S.at[15], R.at[15],
                                     device_id=(left,), **_MESH),
    ]
    for r in b_l:
        r.start()

    quarter(kL.at[2], vL.at[2], 2)
    quarter(kL.at[3], vL.at[3], 3)
    quarter(kR.at[0], vR.at[0], 0)
    quarter(kR.at[1], vR.at[1], 1)
    a_r[2].wait_recv()
    a_r[3].wait_recv()
    quarter(kL.at[0], vL.at[0], 0)
    quarter(kL.at[1], vL.at[1], 1)
    a_l[2].wait_recv()
    a_l[3].wait_recv()
    quarter(kR.at[2], vR.at[2], 2)
    quarter(kR.at[3], vR.at[3], 3)

    b_r[0].wait_recv()
    b_r[1].wait_recv()
    quarter(kO.at[2], vO.at[2], 2)
    b_l[0].wait_recv()
    b_l[1].wait_recv()
    quarter(kO.at[0], vO.at[0], 0)
    b_r[2].wait_recv()
    b_r[3].wait_recv()
    quarter(kO.at[3], vO.at[3], 3)
    b_l[2].wait_recv()
    b_l[3].wait_recv()
    quarter(kO.at[1], vO.at[1], 1)

    for r in a_r + a_l + b_r + b_l:
        r.wait_send()

    def store_body(h, carry):
        out_ref[:, pl.ds(h * DH, DH)] = acc_v[h] / acc_s[h]
        return carry
    lax.fori_loop(0, HQ, store_body, 0)


def _group_rows(t):
    return t.reshape(4, 4, 64, -1).transpose(1, 0, 2, 3).reshape(t.shape)


def kernel(x, Wq, K_ext, V_ext, Wo):
    xg = _group_rows(x[0]).astype(jnp.bfloat16)
    Wqb = Wq.reshape(1024, HQ, DH).transpose(1, 0, 2).astype(jnp.bfloat16)
    Kt = K_ext[0].reshape(4, 4, 64, HQ, DH).transpose(
        1, 3, 0, 2, 4).reshape(4, HQ, QKV, DH).astype(jnp.bfloat16)
    Vt = V_ext[0].reshape(4, 4, 64, HQ, DH).transpose(
        1, 3, 0, 2, 4).reshape(4, HQ, QKV, DH).astype(jnp.bfloat16)

    chunk = pltpu.VMEM((4, HQ, QKV, DH), jnp.bfloat16)
    ctx_g = pl.pallas_call(
        _attn_body,
        out_shape=jax.ShapeDtypeStruct((SQ, HQ * DH), jnp.float32),
        in_specs=[pl.BlockSpec(memory_space=pltpu.VMEM)] * 4,
        out_specs=pl.BlockSpec(memory_space=pltpu.VMEM),
        scratch_shapes=[
            chunk, chunk,
            chunk, chunk,
            chunk, chunk,
            pltpu.VMEM((HQ, SQ, DH), jnp.bfloat16),
            pltpu.VMEM((HQ, SQ, DH), jnp.float32),
            pltpu.VMEM((HQ, SQ, 1), jnp.float32),
            pltpu.SemaphoreType.DMA((16,)),
            pltpu.SemaphoreType.DMA((16,)),
        ],
        compiler_params=pltpu.CompilerParams(collective_id=0),
    )(xg, Wqb, Kt, Vt)

    return (_group_rows(ctx_g) @ Wo)[None]


# device time: 77113 ns/iter; 2.1752x vs baseline; 1.2571x over previous
import jax
import jax.numpy as jnp
from jax import lax
from jax.experimental import pallas as pl
from jax.experimental.pallas import tpu as pltpu

N_DEV = 4
SQ = 1024
SKV = 1024
QKV = SKV // 4
QQ = SQ // 4
HQ = 8
DH = 128
SCALE = 0.08838834764831843

_MESH = dict(device_id_type=pl.DeviceIdType.MESH)


def _attn_body(x_ref, wq_ref, k_ref, v_ref, wo_ref, out_ref,
               kL, vL, kR, vR, kO, vO, q_scr, acc_v, acc_s, ctx_scr, S, R):
    my = lax.axis_index("i")
    left = (my - 1) % N_DEV
    right = (my + 1) % N_DEV

    barrier_sem = pltpu.get_barrier_semaphore()
    for nbr in (left, right):
        pl.semaphore_signal(barrier_sem, inc=1, device_id=(nbr,), **_MESH)
    pl.semaphore_wait(barrier_sem, 2)

    a_r = [
        pltpu.make_async_remote_copy(k_ref.at[pl.ds(2, 2)], kL.at[pl.ds(2, 2)],
                                     S.at[0], R.at[0], device_id=(right,), **_MESH),
        pltpu.make_async_remote_copy(v_ref.at[pl.ds(2, 2)], vL.at[pl.ds(2, 2)],
                                     S.at[1], R.at[1], device_id=(right,), **_MESH),
        pltpu.make_async_remote_copy(k_ref.at[pl.ds(0, 2)], kL.at[pl.ds(0, 2)],
                                     S.at[2], R.at[2], device_id=(right,), **_MESH),
        pltpu.make_async_remote_copy(v_ref.at[pl.ds(0, 2)], vL.at[pl.ds(0, 2)],
                                     S.at[3], R.at[3], device_id=(right,), **_MESH),
    ]
    a_l = [
        pltpu.make_async_remote_copy(k_ref.at[pl.ds(0, 2)], kR.at[pl.ds(0, 2)],
                                     S.at[4], R.at[4], device_id=(left,), **_MESH),
        pltpu.make_async_remote_copy(v_ref.at[pl.ds(0, 2)], vR.at[pl.ds(0, 2)],
                                     S.at[5], R.at[5], device_id=(left,), **_MESH),
        pltpu.make_async_remote_copy(k_ref.at[pl.ds(2, 2)], kR.at[pl.ds(2, 2)],
                                     S.at[6], R.at[6], device_id=(left,), **_MESH),
        pltpu.make_async_remote_copy(v_ref.at[pl.ds(2, 2)], vR.at[pl.ds(2, 2)],
                                     S.at[7], R.at[7], device_id=(left,), **_MESH),
    ]
    for r in a_r + a_l:
        r.start()

    def qproj_body(h, carry):
        q_scr[h] = lax.dot_general(
            x_ref[...], wq_ref[h], (((1,), (0,)), ((), ())),
            preferred_element_type=jnp.float32).astype(jnp.bfloat16)
        return carry
    lax.fori_loop(0, HQ, qproj_body, 0)

    acc_v[...] = jnp.zeros((HQ, SQ, DH), jnp.float32)
    acc_s[...] = jnp.zeros((HQ, SQ, 1), jnp.float32)

    def quarter(kc, vc, r):
        def head_body(h, carry):
            qh = q_scr[h, pl.ds(r * QQ, QQ)]
            s = lax.dot_general(qh, kc[h], (((1,), (1,)), ((), ())),
                                preferred_element_type=jnp.float32) * SCALE
            w = jnp.exp(s)
            pv = lax.dot_general(w.astype(jnp.bfloat16), vc[h],
                                 (((1,), (0,)), ((), ())),
                                 preferred_element_type=jnp.float32)
            row = pl.ds(r * QQ, QQ)
            acc_v[h, row] = acc_v[h, row] + pv
            acc_s[h, row] = acc_s[h, row] + jnp.sum(w, axis=1, keepdims=True)
            return carry
        lax.fori_loop(0, HQ, head_body, 0)

    for r in range(4):
        quarter(k_ref.at[r], v_ref.at[r], r)

    a_r[0].wait_recv()
    a_r[1].wait_recv()
    b_r = [
        pltpu.make_async_remote_copy(kL.at[2], kO.at[2], S.at[8], R.at[8],
                                     device_id=(right,), **_MESH),
        pltpu.make_async_remote_copy(vL.at[2], vO.at[2], S.at[9], R.at[9],
                                     device_id=(right,), **_MESH),
        pltpu.make_async_remote_copy(kL.at[3], kO.at[3], S.at[10], R.at[10],
                                     device_id=(right,), **_MESH),
        pltpu.make_async_remote_copy(vL.at[3], vO.at[3], S.at[11], R.at[11],
                                     device_id=(right,), **_MESH),
    ]
    for r in b_r:
        r.start()
    a_l[0].wait_recv()
    a_l[1].wait_recv()
    b_l = [
        pltpu.make_async_remote_copy(kR.at[0], kO.at[0], S.at[12], R.at[12],
                                     device_id=(left,), **_MESH),
        pltpu.make_async_remote_copy(vR.at[0], vO.at[0], S.at[13], R.at[13],
                                     device_id=(left,), **_MESH),
        pltpu.make_async_remote_copy(kR.at[1], kO.at[1], S.at[14], R.at[14],
                                     device_id=(left,), **_MESH),
        pltpu.make_async_remote_copy(vR.at[1], vO.at[1], S.at[15], R.at[15],
                                     device_id=(left,), **_MESH),
    ]
    for r in b_l:
        r.start()

    quarter(kL.at[2], vL.at[2], 2)
    quarter(kL.at[3], vL.at[3], 3)
    quarter(kR.at[0], vR.at[0], 0)
    quarter(kR.at[1], vR.at[1], 1)
    a_r[2].wait_recv()
    a_r[3].wait_recv()
    quarter(kL.at[0], vL.at[0], 0)
    quarter(kL.at[1], vL.at[1], 1)
    a_l[2].wait_recv()
    a_l[3].wait_recv()
    quarter(kR.at[2], vR.at[2], 2)
    quarter(kR.at[3], vR.at[3], 3)

    def epilogue(r):
        row = pl.ds(r * QQ, QQ)
        for h in range(HQ):
            ctx_scr[:, h * DH:(h + 1) * DH] = acc_v[h, row] / acc_s[h, row]
        res = lax.dot_general(ctx_scr[...].astype(jnp.bfloat16), wo_ref[...],
                              (((1,), (0,)), ((), ())),
                              preferred_element_type=jnp.float32)
        for g in range(4):
            out_ref[pl.ds((g * 4 + r) * 64, 64), :] = res[g * 64:(g + 1) * 64, :]

    b_r[0].wait_recv()
    b_r[1].wait_recv()
    quarter(kO.at[2], vO.at[2], 2)
    epilogue(2)
    b_l[0].wait_recv()
    b_l[1].wait_recv()
    quarter(kO.at[0], vO.at[0], 0)
    epilogue(0)
    b_r[2].wait_recv()
    b_r[3].wait_recv()
    quarter(kO.at[3], vO.at[3], 3)
    epilogue(3)
    b_l[2].wait_recv()
    b_l[3].wait_recv()
    quarter(kO.at[1], vO.at[1], 1)
    epilogue(1)

    for r in a_r + a_l + b_r + b_l:
        r.wait_send()


def _group_rows(t):
    return t.reshape(4, 4, 64, -1).transpose(1, 0, 2, 3).reshape(t.shape)


def kernel(x, Wq, K_ext, V_ext, Wo):
    xg = _group_rows(x[0]).astype(jnp.bfloat16)
    Wqb = Wq.reshape(1024, HQ, DH).transpose(1, 0, 2).astype(jnp.bfloat16)
    Kt = K_ext[0].reshape(4, 4, 64, HQ, DH).transpose(
        1, 3, 0, 2, 4).reshape(4, HQ, QKV, DH).astype(jnp.bfloat16)
    Vt = V_ext[0].reshape(4, 4, 64, HQ, DH).transpose(
        1, 3, 0, 2, 4).reshape(4, HQ, QKV, DH).astype(jnp.bfloat16)

    chunk = pltpu.VMEM((4, HQ, QKV, DH), jnp.bfloat16)
    out = pl.pallas_call(
        _attn_body,
        out_shape=jax.ShapeDtypeStruct((SQ, HQ * DH), jnp.float32),
        in_specs=[pl.BlockSpec(memory_space=pltpu.VMEM)] * 5,
        out_specs=pl.BlockSpec(memory_space=pltpu.VMEM),
        scratch_shapes=[
            chunk, chunk,
            chunk, chunk,
            chunk, chunk,
            pltpu.VMEM((HQ, SQ, DH), jnp.bfloat16),
            pltpu.VMEM((HQ, SQ, DH), jnp.float32),
            pltpu.VMEM((HQ, SQ, 1), jnp.float32),
            pltpu.VMEM((QQ, HQ * DH), jnp.float32),
            pltpu.SemaphoreType.DMA((16,)),
            pltpu.SemaphoreType.DMA((16,)),
        ],
        compiler_params=pltpu.CompilerParams(collective_id=0),
    )(xg, Wqb, Kt, Vt, Wo.astype(jnp.bfloat16))

    return out[None]
